# baseline (speedup 1.0000x reference)
import functools

import jax
import jax.numpy as jnp
from jax import lax
from jax.experimental import pallas as pl
from jax.experimental.pallas import tpu as pltpu
from jax.experimental.pallas import tpu_sc as plsc

B = 16384
VOCABN = 100000
SLICES = (10240, 6144)
OFFS = (0, 10240)
SEQ = 20
D = 128
HIDDEN = 1024

NC = 2
NS = 16
NW = NC * NS

CHUNK_ROWS = 8
GROWS = SEQ * CHUNK_ROWS
NBUF = 4


def _sc_body(off, bpw, x_hbm, table_hbm, out_hbm,
             idx_v, rows, outs, sems, semo):
    chunks = bpw // CHUNK_ROWS
    quads = chunks // NBUF
    wid = lax.axis_index("s") * NC + lax.axis_index("c")
    obase = wid * bpw
    pltpu.sync_copy(
        x_hbm.at[pl.ds(off + wid * bpw, bpw), :], idx_v)

    def fire_gather(i, b):
        for r in range(CHUNK_ROWS):
            pltpu.async_copy(table_hbm.at[idx_v.at[i * CHUNK_ROWS + r]],
                             rows[b].at[pl.ds(r * SEQ, SEQ), :], sems[b])

    def drain_gather(b):
        pltpu.make_async_copy(
            table_hbm.at[pl.ds(0, GROWS), :], rows[b], sems[b]).wait()

    def reduce_chunk(b):
        buf = rows[b]
        obuf = outs[b]

        def row_body(r, c):
            base = r * SEQ
            for h in range(D // 16):
                sl = pl.ds(h * 16, 16)
                vals = [buf[base + j, sl] for j in range(SEQ)]
                while len(vals) > 1:
                    vals = [a + b2 for a, b2 in zip(vals[::2], vals[1::2])] \
                        + ([vals[-1]] if len(vals) % 2 else [])
                obuf[r, sl] = vals[0] * jnp.float32(1.0 / SEQ)
            return c
        lax.fori_loop(0, CHUNK_ROWS, row_body, 0)

    def fire_out(i, b):
        pltpu.async_copy(
            outs[b], out_hbm.at[pl.ds(obase + i * CHUNK_ROWS, CHUNK_ROWS), :],
            semo[b])

    def drain_out(b):
        pltpu.make_async_copy(
            outs[b], out_hbm.at[pl.ds(obase, CHUNK_ROWS), :], semo[b]).wait()

    for b in range(NBUF - 1):
        fire_gather(b, b)

    def quad_body(g, carry):
        i0 = g * NBUF
        for k in range(NBUF):
            @pl.when(i0 + k + NBUF - 1 < chunks)
            def _():
                fire_gather(i0 + k + NBUF - 1, (k + NBUF - 1) % NBUF)
            drain_gather(k)

            @pl.when(g > 0)
            def _():
                drain_out(k)
            reduce_chunk(k)
            fire_out(i0 + k, k)
        return carry

    lax.fori_loop(0, quads, quad_body, 0)
    for b in range(NBUF):
        drain_out(b)


def _make_sc(off, rows_total):
    bpw = rows_total // NW
    return functools.partial(
        pl.kernel,
        out_type=jax.ShapeDtypeStruct((rows_total, D), jnp.float32),
        mesh=plsc.VectorSubcoreMesh(core_axis_name="c", subcore_axis_name="s"),
        scratch_types=[
            pltpu.VMEM((bpw, SEQ), jnp.int32),
            [pltpu.VMEM((GROWS, D), jnp.float32) for _ in range(NBUF)],
            [pltpu.VMEM((CHUNK_ROWS, D), jnp.float32) for _ in range(NBUF)],
            [pltpu.SemaphoreType.DMA for _ in range(NBUF)],
            [pltpu.SemaphoreType.DMA for _ in range(NBUF)],
        ],
    )(functools.partial(_sc_body, off, bpw))


_sc_gather_mean = [_make_sc(OFFS[i], SLICES[i]) for i in range(2)]


def _mlp_body(m_ref, w1_ref, b1_ref, w2_ref, b2_ref, o_ref):
    m = m_ref[...].astype(jnp.bfloat16)
    h = lax.dot_general(m, w1_ref[...].astype(jnp.bfloat16),
                        (((1,), (1,)), ((), ())),
                        preferred_element_type=jnp.float32)
    h = jnp.maximum(h + b1_ref[...], 0.0)
    o = lax.dot_general(h, w2_ref[...], (((1,), (1,)), ((), ())),
                        preferred_element_type=jnp.float32)
    o = jnp.maximum(o + b2_ref[...], 0.0)
    mx = jnp.max(o, axis=1, keepdims=True)
    lse = mx + jnp.log(jnp.sum(jnp.exp(o - mx), axis=1, keepdims=True))
    o_ref[...] = o - lse


def _mlp(m, W1, b1, W2, b2, bs=2048):
    rows_total = m.shape[0]
    grid = (rows_total // bs,)
    return pl.pallas_call(
        _mlp_body,
        grid=grid,
        in_specs=[
            pl.BlockSpec((bs, D), lambda i: (i, 0)),
            pl.BlockSpec((HIDDEN, D), lambda i: (0, 0)),
            pl.BlockSpec((1, HIDDEN), lambda i: (0, 0)),
            pl.BlockSpec((2, HIDDEN), lambda i: (0, 0)),
            pl.BlockSpec((1, 2), lambda i: (0, 0)),
        ],
        out_specs=pl.BlockSpec((bs, 2), lambda i: (i, 0)),
        out_shape=jax.ShapeDtypeStruct((rows_total, 2), jnp.float32),
    )(m, W1, b1, W2, b2)


def kernel(x, table, W1, b1, W2, b2):
    xi = x.astype(jnp.int32)
    b1r = b1.reshape(1, HIDDEN)
    b2r = b2.reshape(1, 2)
    ms = [_sc_gather_mean[i](xi, table) for i in range(2)]
    outs = [_mlp(m, W1, b1r, W2, b2r) for m in ms]
    return jnp.concatenate(outs, axis=0)


# speedup vs baseline: 1.0323x; 1.0323x over previous
import functools

import jax
import jax.numpy as jnp
from jax import lax
from jax.experimental import pallas as pl
from jax.experimental.pallas import tpu as pltpu
from jax.experimental.pallas import tpu_sc as plsc

B = 16384
VOCABN = 100000
SLICES = (8192, 8192)
OFFS = (0, 8192)
SEQ = 20
D = 128
HIDDEN = 1024

NC = 2
NS = 16
NW = NC * NS

CHUNK_ROWS = 8
GROWS = SEQ * CHUNK_ROWS
NBUF = 4


def _sc_body(off, bpw, x_hbm, table_hbm, out_hbm,
             idx_v, rows, outs, sems, semo):
    chunks = bpw // CHUNK_ROWS
    quads = chunks // NBUF
    wid = lax.axis_index("s") * NC + lax.axis_index("c")
    obase = wid * bpw
    pltpu.sync_copy(
        x_hbm.at[pl.ds(off + wid * bpw, bpw), :], idx_v)

    def fire_gather(i, b):
        for r in range(CHUNK_ROWS):
            pltpu.async_copy(table_hbm.at[idx_v.at[i * CHUNK_ROWS + r]],
                             rows[b].at[pl.ds(r * SEQ, SEQ), :], sems[b])

    def drain_gather(b):
        pltpu.make_async_copy(
            table_hbm.at[pl.ds(0, GROWS), :], rows[b], sems[b]).wait()

    def reduce_chunk(b):
        buf = rows[b]
        obuf = outs[b]

        def row_body(r, c):
            base = r * SEQ
            for h in range(D // 16):
                sl = pl.ds(h * 16, 16)
                vals = [buf[base + j, sl] for j in range(SEQ)]
                while len(vals) > 1:
                    vals = [a + b2 for a, b2 in zip(vals[::2], vals[1::2])] \
                        + ([vals[-1]] if len(vals) % 2 else [])
                obuf[r, sl] = vals[0] * jnp.float32(1.0 / SEQ)
            return c
        lax.fori_loop(0, CHUNK_ROWS, row_body, 0)

    def fire_out(i, b):
        pltpu.async_copy(
            outs[b], out_hbm.at[pl.ds(obase + i * CHUNK_ROWS, CHUNK_ROWS), :],
            semo[b])

    def drain_out(b):
        pltpu.make_async_copy(
            outs[b], out_hbm.at[pl.ds(obase, CHUNK_ROWS), :], semo[b]).wait()

    for b in range(NBUF - 1):
        fire_gather(b, b)

    def quad_body(g, carry):
        i0 = g * NBUF
        for k in range(NBUF):
            @pl.when(i0 + k + NBUF - 1 < chunks)
            def _():
                fire_gather(i0 + k + NBUF - 1, (k + NBUF - 1) % NBUF)
            drain_gather(k)

            @pl.when(g > 0)
            def _():
                drain_out(k)
            reduce_chunk(k)
            fire_out(i0 + k, k)
        return carry

    lax.fori_loop(0, quads, quad_body, 0)
    for b in range(NBUF):
        drain_out(b)


def _make_sc(off, rows_total):
    bpw = rows_total // NW
    return functools.partial(
        pl.kernel,
        out_type=jax.ShapeDtypeStruct((rows_total, D), jnp.float32),
        mesh=plsc.VectorSubcoreMesh(core_axis_name="c", subcore_axis_name="s"),
        scratch_types=[
            pltpu.VMEM((bpw, SEQ), jnp.int32),
            [pltpu.VMEM((GROWS, D), jnp.float32) for _ in range(NBUF)],
            [pltpu.VMEM((CHUNK_ROWS, D), jnp.float32) for _ in range(NBUF)],
            [pltpu.SemaphoreType.DMA for _ in range(NBUF)],
            [pltpu.SemaphoreType.DMA for _ in range(NBUF)],
        ],
    )(functools.partial(_sc_body, off, bpw))


_sc_gather_mean = [_make_sc(OFFS[i], SLICES[i]) for i in range(2)]


def _mlp_body(m_ref, w1_ref, b1_ref, w2_ref, b2_ref, o_ref):
    m = m_ref[...].astype(jnp.bfloat16)
    h = lax.dot_general(m, w1_ref[...].astype(jnp.bfloat16),
                        (((1,), (1,)), ((), ())),
                        preferred_element_type=jnp.float32)
    h = jnp.maximum(h + b1_ref[...], 0.0)
    o = lax.dot_general(h, w2_ref[...], (((1,), (1,)), ((), ())),
                        preferred_element_type=jnp.float32)
    o = jnp.maximum(o + b2_ref[...], 0.0)
    mx = jnp.max(o, axis=1, keepdims=True)
    lse = mx + jnp.log(jnp.sum(jnp.exp(o - mx), axis=1, keepdims=True))
    o_ref[...] = o - lse


def _mlp(m, W1, b1, W2, b2, bs=2048):
    rows_total = m.shape[0]
    grid = (rows_total // bs,)
    return pl.pallas_call(
        _mlp_body,
        grid=grid,
        in_specs=[
            pl.BlockSpec((bs, D), lambda i: (i, 0)),
            pl.BlockSpec((HIDDEN, D), lambda i: (0, 0)),
            pl.BlockSpec((1, HIDDEN), lambda i: (0, 0)),
            pl.BlockSpec((2, HIDDEN), lambda i: (0, 0)),
            pl.BlockSpec((1, 2), lambda i: (0, 0)),
        ],
        out_specs=pl.BlockSpec((bs, 2), lambda i: (i, 0)),
        out_shape=jax.ShapeDtypeStruct((rows_total, 2), jnp.float32),
    )(m, W1, b1, W2, b2)


def kernel(x, table, W1, b1, W2, b2):
    xi = x.astype(jnp.int32)
    b1r = b1.reshape(1, HIDDEN)
    b2r = b2.reshape(1, 2)
    ms = [_sc_gather_mean[i](xi, table) for i in range(2)]
    outs = [_mlp(m, W1, b1r, W2, b2r) for m in ms]
    return jnp.concatenate(outs, axis=0)


# speedup vs baseline: 1.0364x; 1.0040x over previous
import functools

import jax
import jax.numpy as jnp
from jax import lax
from jax.experimental import pallas as pl
from jax.experimental.pallas import tpu as pltpu
from jax.experimental.pallas import tpu_sc as plsc

B = 16384
VOCABN = 100000
SLICES = (8192, 8192)
OFFS = (0, 8192)
SEQ = 20
D = 128
HIDDEN = 1024

NC = 2
NS = 16
NW = NC * NS

CHUNK_ROWS = 8
GROWS = SEQ * CHUNK_ROWS
NBUF = 4


def _sc_body(off, bpw, x_hbm, table_hbm, out_hbm,
             idx_v, rows, outs, sems, semo):
    chunks = bpw // CHUNK_ROWS
    quads = chunks // NBUF
    wid = lax.axis_index("s") * NC + lax.axis_index("c")
    obase = wid * bpw
    pltpu.sync_copy(
        x_hbm.at[pl.ds(off + wid * bpw, bpw), :], idx_v)

    def fire_gather(i, b):
        for r in range(CHUNK_ROWS):
            pltpu.async_copy(table_hbm.at[idx_v.at[i * CHUNK_ROWS + r]],
                             rows[b].at[pl.ds(r * SEQ, SEQ), :], sems[b])

    def drain_gather(b):
        pltpu.make_async_copy(
            table_hbm.at[pl.ds(0, GROWS), :], rows[b], sems[b]).wait()

    def reduce_chunk(b):
        buf = rows[b]
        obuf = outs[b]

        def row_body(r, c):
            base = r * SEQ
            for h in range(D // 16):
                sl = pl.ds(h * 16, 16)
                vals = [buf[base + j, sl] for j in range(SEQ)]
                while len(vals) > 1:
                    vals = [a + b2 for a, b2 in zip(vals[::2], vals[1::2])] \
                        + ([vals[-1]] if len(vals) % 2 else [])
                obuf[r, sl] = vals[0] * jnp.float32(1.0 / SEQ)
            return c
        lax.fori_loop(0, CHUNK_ROWS, row_body, 0)

    def fire_out(i, b):
        pltpu.async_copy(
            outs[b], out_hbm.at[pl.ds(obase + i * CHUNK_ROWS, CHUNK_ROWS), :],
            semo[b])

    def drain_out(b):
        pltpu.make_async_copy(
            outs[b], out_hbm.at[pl.ds(obase, CHUNK_ROWS), :], semo[b]).wait()

    for b in range(NBUF - 1):
        fire_gather(b, b)

    def quad_body(g, carry):
        i0 = g * NBUF
        for k in range(NBUF):
            @pl.when(i0 + k + NBUF - 1 < chunks)
            def _():
                fire_gather(i0 + k + NBUF - 1, (k + NBUF - 1) % NBUF)
            drain_gather(k)

            @pl.when(g > 0)
            def _():
                drain_out(k)
            reduce_chunk(k)
            fire_out(i0 + k, k)
        return carry

    lax.fori_loop(0, quads, quad_body, 0)
    for b in range(NBUF):
        drain_out(b)


def _make_sc(off, rows_total):
    bpw = rows_total // NW
    return functools.partial(
        pl.kernel,
        out_type=jax.ShapeDtypeStruct((rows_total, D), jnp.float32),
        mesh=plsc.VectorSubcoreMesh(core_axis_name="c", subcore_axis_name="s"),
        scratch_types=[
            pltpu.VMEM((bpw, SEQ), jnp.int32),
            [pltpu.VMEM((GROWS, D), jnp.float32) for _ in range(NBUF)],
            [pltpu.VMEM((CHUNK_ROWS, D), jnp.float32) for _ in range(NBUF)],
            [pltpu.SemaphoreType.DMA for _ in range(NBUF)],
            [pltpu.SemaphoreType.DMA for _ in range(NBUF)],
        ],
    )(functools.partial(_sc_body, off, bpw))


_sc_gather_mean = [_make_sc(OFFS[i], SLICES[i]) for i in range(2)]


def _mlp_body(m_ref, w1_ref, b1_ref, w2_ref, b2_ref, o_ref):
    m = m_ref[...]
    h = lax.dot_general(m, w1_ref[...], (((1,), (1,)), ((), ())),
                        preferred_element_type=jnp.float32)
    h = jnp.maximum(h + b1_ref[...], 0.0)
    o = lax.dot_general(h, w2_ref[...], (((1,), (1,)), ((), ())),
                        preferred_element_type=jnp.float32)
    o = jnp.maximum(o + b2_ref[...], 0.0)
    mx = jnp.max(o, axis=1, keepdims=True)
    lse = mx + jnp.log(jnp.sum(jnp.exp(o - mx), axis=1, keepdims=True))
    o_ref[...] = o - lse


def _mlp(m, W1, b1, W2, b2, bs=4096):
    rows_total = m.shape[0]
    grid = (rows_total // bs,)
    return pl.pallas_call(
        _mlp_body,
        grid=grid,
        in_specs=[
            pl.BlockSpec((bs, D), lambda i: (i, 0)),
            pl.BlockSpec((HIDDEN, D), lambda i: (0, 0)),
            pl.BlockSpec((1, HIDDEN), lambda i: (0, 0)),
            pl.BlockSpec((2, HIDDEN), lambda i: (0, 0)),
            pl.BlockSpec((1, 2), lambda i: (0, 0)),
        ],
        out_specs=pl.BlockSpec((bs, 2), lambda i: (i, 0)),
        out_shape=jax.ShapeDtypeStruct((rows_total, 2), jnp.float32),
    )(m, W1, b1, W2, b2)


def kernel(x, table, W1, b1, W2, b2):
    xi = x.astype(jnp.int32)
    b1r = b1.reshape(1, HIDDEN)
    b2r = b2.reshape(1, 2)
    ms = [_sc_gather_mean[i](xi, table) for i in range(2)]
    outs = [_mlp(m, W1, b1r, W2, b2r) for m in ms]
    return jnp.concatenate(outs, axis=0)


# speedup vs baseline: 1.0514x; 1.0144x over previous
import functools

import jax
import jax.numpy as jnp
from jax import lax
from jax.experimental import pallas as pl
from jax.experimental.pallas import tpu as pltpu
from jax.experimental.pallas import tpu_sc as plsc

B = 16384
VOCABN = 100000
SLICES = (8192, 8192)
OFFS = (0, 8192)
SEQ = 20
D = 128
HIDDEN = 1024

NC = 2
NS = 16
NW = NC * NS

CHUNK_ROWS = 4
GROWS = SEQ * CHUNK_ROWS
NBUF = 4


def _sc_body(off, bpw, x_hbm, table_hbm, out_hbm,
             idx_v, rows, outs, sems, semo):
    chunks = bpw // CHUNK_ROWS
    quads = chunks // NBUF
    wid = lax.axis_index("s") * NC + lax.axis_index("c")
    obase = wid * bpw
    pltpu.sync_copy(
        x_hbm.at[pl.ds(off + wid * bpw, bpw), :], idx_v)

    def fire_gather(i, b):
        for r in range(CHUNK_ROWS):
            pltpu.async_copy(table_hbm.at[idx_v.at[i * CHUNK_ROWS + r]],
                             rows[b].at[pl.ds(r * SEQ, SEQ), :], sems[b])

    def drain_gather(b):
        pltpu.make_async_copy(
            table_hbm.at[pl.ds(0, GROWS), :], rows[b], sems[b]).wait()

    def reduce_chunk(b):
        buf = rows[b]
        obuf = outs[b]

        def row_body(r, c):
            base = r * SEQ
            for h in range(D // 16):
                sl = pl.ds(h * 16, 16)
                vals = [buf[base + j, sl] for j in range(SEQ)]
                while len(vals) > 1:
                    vals = [a + b2 for a, b2 in zip(vals[::2], vals[1::2])] \
                        + ([vals[-1]] if len(vals) % 2 else [])
                obuf[r, sl] = vals[0] * jnp.float32(1.0 / SEQ)
            return c
        lax.fori_loop(0, CHUNK_ROWS, row_body, 0)

    def fire_out(i, b):
        pltpu.async_copy(
            outs[b], out_hbm.at[pl.ds(obase + i * CHUNK_ROWS, CHUNK_ROWS), :],
            semo[b])

    def drain_out(b):
        pltpu.make_async_copy(
            outs[b], out_hbm.at[pl.ds(obase, CHUNK_ROWS), :], semo[b]).wait()

    for b in range(NBUF - 1):
        fire_gather(b, b)

    def quad_body(g, carry):
        i0 = g * NBUF
        for k in range(NBUF):
            @pl.when(i0 + k + NBUF - 1 < chunks)
            def _():
                fire_gather(i0 + k + NBUF - 1, (k + NBUF - 1) % NBUF)
            drain_gather(k)

            @pl.when(g > 0)
            def _():
                drain_out(k)
            reduce_chunk(k)
            fire_out(i0 + k, k)
        return carry

    lax.fori_loop(0, quads, quad_body, 0)
    for b in range(NBUF):
        drain_out(b)


def _make_sc(off, rows_total):
    bpw = rows_total // NW
    return functools.partial(
        pl.kernel,
        out_type=jax.ShapeDtypeStruct((rows_total, D), jnp.float32),
        mesh=plsc.VectorSubcoreMesh(core_axis_name="c", subcore_axis_name="s"),
        scratch_types=[
            pltpu.VMEM((bpw, SEQ), jnp.int32),
            [pltpu.VMEM((GROWS, D), jnp.float32) for _ in range(NBUF)],
            [pltpu.VMEM((CHUNK_ROWS, D), jnp.float32) for _ in range(NBUF)],
            [pltpu.SemaphoreType.DMA for _ in range(NBUF)],
            [pltpu.SemaphoreType.DMA for _ in range(NBUF)],
        ],
    )(functools.partial(_sc_body, off, bpw))


_sc_gather_mean = [_make_sc(OFFS[i], SLICES[i]) for i in range(2)]


def _mlp_body(m_ref, w1_ref, b1_ref, w2_ref, b2_ref, o_ref):
    m = m_ref[...]
    h = lax.dot_general(m, w1_ref[...], (((1,), (1,)), ((), ())),
                        preferred_element_type=jnp.float32)
    h = jnp.maximum(h + b1_ref[...], 0.0)
    o = lax.dot_general(h, w2_ref[...], (((1,), (1,)), ((), ())),
                        preferred_element_type=jnp.float32)
    o = jnp.maximum(o + b2_ref[...], 0.0)
    mx = jnp.max(o, axis=1, keepdims=True)
    lse = mx + jnp.log(jnp.sum(jnp.exp(o - mx), axis=1, keepdims=True))
    o_ref[...] = o - lse


def _mlp(m, W1, b1, W2, b2, bs=4096):
    rows_total = m.shape[0]
    grid = (rows_total // bs,)
    return pl.pallas_call(
        _mlp_body,
        grid=grid,
        in_specs=[
            pl.BlockSpec((bs, D), lambda i: (i, 0)),
            pl.BlockSpec((HIDDEN, D), lambda i: (0, 0)),
            pl.BlockSpec((1, HIDDEN), lambda i: (0, 0)),
            pl.BlockSpec((2, HIDDEN), lambda i: (0, 0)),
            pl.BlockSpec((1, 2), lambda i: (0, 0)),
        ],
        out_specs=pl.BlockSpec((bs, 2), lambda i: (i, 0)),
        out_shape=jax.ShapeDtypeStruct((rows_total, 2), jnp.float32),
    )(m, W1, b1, W2, b2)


def kernel(x, table, W1, b1, W2, b2):
    xi = x.astype(jnp.int32)
    b1r = b1.reshape(1, HIDDEN)
    b2r = b2.reshape(1, 2)
    ms = [_sc_gather_mean[i](xi, table) for i in range(2)]
    outs = [_mlp(m, W1, b1r, W2, b2r) for m in ms]
    return jnp.concatenate(outs, axis=0)
